# Initial kernel scaffold; baseline (speedup 1.0000x reference)
import functools

import jax
import jax.numpy as jnp
from jax import lax
from jax.experimental import pallas as pl
from jax.experimental.pallas import tpu as pltpu
from jax.experimental.pallas import tpu_sc as plsc

_N = 10000
_E = 320000
_G = 64
_D1 = 112
_D2 = 32

_NC, _NS = 2, 16
_NW = _NC * _NS
_CH = 128
_NCHUNKS = _E // _CH
_BASE_CHUNKS = _NCHUNKS // _NW
_EXTRA = _NCHUNKS % _NW
_ROWS_PER = _N // _NS
_ZR = 125

_BM = 1000


def _sc_mean_agg(table, edge_index, d):
    mesh = plsc.VectorSubcoreMesh(core_axis_name="c", subcore_axis_name="s")

    @functools.partial(
        pl.kernel,
        mesh=mesh,
        out_type=jax.ShapeDtypeStruct((_NC, _N, d), jnp.float32),
        scratch_types=[
            pltpu.VMEM((_CH,), jnp.int32),
            pltpu.VMEM((_CH,), jnp.int32),
            pltpu.VMEM((_CH, d), jnp.float32),
            pltpu.VMEM((_ZR, d), jnp.float32),
            pltpu.VMEM_SHARED((_N, d), jnp.float32),
            pltpu.SemaphoreType.DMA,
        ],
    )
    def k(table_hbm, ei_hbm, out_hbm, src_v, dst_v, rows_v, z_v, acc_sh, sem):
        cid = lax.axis_index("c")
        sid = lax.axis_index("s")
        w = cid * _NS + sid

        @pl.loop(0, _ZR)
        def _(r):
            for j in range(d // 16):
                z_v[r, pl.ds(16 * j, 16)] = jnp.zeros((16,), jnp.float32)

        row0 = sid * _ROWS_PER

        @pl.loop(0, _ROWS_PER // _ZR)
        def _(t):
            pltpu.sync_copy(z_v, acc_sh.at[pl.ds(row0 + t * _ZR, _ZR)])

        plsc.subcore_barrier()

        def do_chunk(chunk):
            base = chunk * _CH
            pltpu.sync_copy(ei_hbm.at[0, pl.ds(base, _CH)], src_v)
            pltpu.sync_copy(ei_hbm.at[1, pl.ds(base, _CH)], dst_v)
            pltpu.async_copy(table_hbm.at[src_v], rows_v, sem).wait()
            pltpu.sync_copy(rows_v, acc_sh.at[dst_v], add=True)

        @pl.loop(0, _BASE_CHUNKS)
        def _(j):
            do_chunk(w + j * _NW)

        @pl.when(w < _EXTRA)
        def _():
            do_chunk(w + _BASE_CHUNKS * _NW)

        plsc.subcore_barrier()
        pltpu.sync_copy(acc_sh.at[pl.ds(row0, _ROWS_PER)],
                        out_hbm.at[cid, pl.ds(row0, _ROWS_PER)])

    return k(table, edge_index)


def _t1(x, w1p):
    def body(x_ref, w_ref, o_ref):
        col = lax.broadcasted_iota(jnp.int32, (_BM, _D1), 1)
        acc = jnp.dot(x_ref[...], w_ref[...],
                      preferred_element_type=jnp.float32)
        o_ref[...] = acc + (col == 0).astype(jnp.float32)

    return pl.pallas_call(
        body,
        grid=(_N // _BM,),
        in_specs=[pl.BlockSpec((_BM, 128), lambda i: (i, 0)),
                  pl.BlockSpec((128, _D1), lambda i: (0, 0))],
        out_specs=pl.BlockSpec((_BM, _D1), lambda i: (i, 0)),
        out_shape=jax.ShapeDtypeStruct((_N, _D1), jnp.float32),
    )(x, w1p)


def _t2(p1, b1p, w2p):
    def body(p_ref, b_ref, w_ref, o_ref):
        pa = p_ref[0] + p_ref[1]
        deg = jnp.maximum(pa[:, 0:1], 1.0)
        h = jnp.maximum(pa / deg + b_ref[...], 0.0)
        col = lax.broadcasted_iota(jnp.int32, (_BM, _D2), 1)
        o_ref[...] = (jnp.dot(h, w_ref[...],
                              preferred_element_type=jnp.float32)
                      + (col == 0).astype(jnp.float32))

    return pl.pallas_call(
        body,
        grid=(_N // _BM,),
        in_specs=[pl.BlockSpec((_NC, _BM, _D1), lambda i: (0, i, 0)),
                  pl.BlockSpec((1, _D1), lambda i: (0, 0)),
                  pl.BlockSpec((_D1, _D2), lambda i: (0, 0))],
        out_specs=pl.BlockSpec((_BM, _D2), lambda i: (i, 0)),
        out_shape=jax.ShapeDtypeStruct((_N, _D2), jnp.float32),
    )(p1, b1p, w2p)


def _bn(z, g, b):
    m = jnp.mean(z, axis=0, keepdims=True)
    v = jnp.mean((z - m) ** 2, axis=0, keepdims=True)
    return g * (z - m) / jnp.sqrt(v + 1e-5) + b


def _t3(p2, gids, b2p, self_feat, fc1_w, fc1_b, bn1_g, bn1_b,
        fc2_w, fc2_b, bn2_g, bn2_b, fc3_w, fc3_b):
    steps = _N // _BM

    def body(p_ref, g_ref, b2_ref, sf_ref, w1_ref, w1b_ref, g1_ref, bb1_ref,
             w2_ref, w2b_ref, g2_ref, bb2_ref, w3_ref, w3b_ref, o_ref,
             acc_ref):
        i = pl.program_id(0)

        @pl.when(i == 0)
        def _():
            acc_ref[...] = jnp.zeros_like(acc_ref)

        pa = p_ref[0] + p_ref[1]
        deg = jnp.maximum(pa[:, 0:1], 1.0)
        h2 = jnp.maximum(pa / deg + b2_ref[...], 0.0)
        col = lax.broadcasted_iota(jnp.int32, (_BM, _D2), 1)
        h2 = h2 + (col == 0).astype(jnp.float32)
        seg = lax.broadcasted_iota(jnp.int32, (_G, _BM), 0)
        onehot = (g_ref[...] == seg).astype(jnp.float32)
        acc_ref[...] += jnp.dot(onehot, h2,
                                preferred_element_type=jnp.float32)

        @pl.when(i == steps - 1)
        def _():
            acc = acc_ref[...]
            cnt = jnp.maximum(acc[:, 0:1], 1.0)
            hg = acc[:, 1:21] / cnt
            c1 = jnp.concatenate([hg, sf_ref[...]], axis=1)
            z = jnp.dot(c1, w1_ref[...],
                        preferred_element_type=jnp.float32) + w1b_ref[...]
            o1 = jnp.maximum(_bn(z, g1_ref[...], bb1_ref[...]), 0.0)
            c2 = jnp.concatenate([o1, sf_ref[...]], axis=1)
            z2 = jnp.dot(c2, w2_ref[...],
                         preferred_element_type=jnp.float32) + w2b_ref[...]
            o2 = jnp.maximum(_bn(z2, g2_ref[...], bb2_ref[...]), 0.0)
            o_ref[...] = jnp.dot(o2, w3_ref[...],
                                 preferred_element_type=jnp.float32) + w3b_ref[...]

    def full(shape):
        return pl.BlockSpec(shape, lambda i: tuple(0 for _ in shape))

    return pl.pallas_call(
        body,
        grid=(steps,),
        in_specs=[pl.BlockSpec((_NC, _BM, _D2), lambda i: (0, i, 0)),
                  pl.BlockSpec((1, _BM), lambda i: (i, 0)),
                  full((1, _D2)),
                  full((_G, 16)),
                  full((36, 256)), full((1, 256)), full((1, 256)), full((1, 256)),
                  full((272, 32)), full((1, 32)), full((1, 32)), full((1, 32)),
                  full((32, 10)), full((1, 10))],
        out_specs=pl.BlockSpec((_G, 10), lambda i: (0, 0)),
        out_shape=jax.ShapeDtypeStruct((_G, 10), jnp.float32),
        scratch_shapes=[pltpu.VMEM((_G, _D2), jnp.float32)],
    )(p2, gids, b2p, self_feat, fc1_w, fc1_b, bn1_g, bn1_b,
      fc2_w, fc2_b, bn2_g, bn2_b, fc3_w, fc3_b)


def kernel(x, edge_index, graph_ids, self_feat, W1, b1, W2, b2,
           fc1_w, fc1_b, bn1_g, bn1_b, fc2_w, fc2_b, bn2_g, bn2_b,
           fc3_w, fc3_b):
    f32 = jnp.float32
    w1p = jnp.zeros((128, _D1), f32).at[:, 1:101].set(W1)
    b1p = jnp.zeros((1, _D1), f32).at[0, 0].set(-1.0).at[0, 1:101].set(b1)
    w2p = jnp.zeros((_D1, _D2), f32).at[1:101, 1:21].set(W2)
    b2p = jnp.zeros((1, _D2), f32).at[0, 0].set(-1.0).at[0, 1:21].set(b2)

    t1 = _t1(x, w1p)
    p1 = _sc_mean_agg(t1, edge_index, _D1)
    t2 = _t2(p1, b1p, w2p)
    p2 = _sc_mean_agg(t2, edge_index, _D2)
    gids = graph_ids.reshape(_N // _BM, _BM)
    return _t3(p2, gids, b2p, self_feat,
               fc1_w, fc1_b.reshape(1, -1), bn1_g.reshape(1, -1),
               bn1_b.reshape(1, -1), fc2_w, fc2_b.reshape(1, -1),
               bn2_g.reshape(1, -1), bn2_b.reshape(1, -1),
               fc3_w, fc3_b.reshape(1, -1))


# speedup vs baseline: 8.3103x; 8.3103x over previous
import functools

import jax
import jax.numpy as jnp
from jax import lax
from jax.experimental import pallas as pl
from jax.experimental.pallas import tpu as pltpu
from jax.experimental.pallas import tpu_sc as plsc

_N = 10000
_E = 320000
_G = 64
_D1 = 112
_D2 = 32

_NC, _NS = 2, 16
_NW = _NC * _NS
_CH = 128
_NCHUNKS = _E // _CH
_BASE_CHUNKS = _NCHUNKS // _NW
_EXTRA = _NCHUNKS % _NW
_ROWS_A = 632
_ROWS_B = 624
_ZR = 104

_BM = 1000


def _sc_mean_agg(table, edge_index, d):
    mesh = plsc.VectorSubcoreMesh(core_axis_name="c", subcore_axis_name="s")

    @functools.partial(
        pl.kernel,
        mesh=mesh,
        compiler_params=pltpu.CompilerParams(use_tc_tiling_on_sc=False),
        out_type=jax.ShapeDtypeStruct((_NC, _N, d), jnp.float32),
        scratch_types=[
            pltpu.VMEM((_CH,), jnp.int32),
            pltpu.VMEM((_CH,), jnp.int32),
            pltpu.VMEM((_CH, d), jnp.float32),
            pltpu.VMEM((_ZR, d), jnp.float32),
            pltpu.VMEM_SHARED((_N, d), jnp.float32),
            pltpu.SemaphoreType.DMA,
        ],
    )
    def k(table_hbm, ei_hbm, out_hbm, src_v, dst_v, rows_v, z_v, acc_sh, sem):
        cid = lax.axis_index("c")
        sid = lax.axis_index("s")
        w = cid * _NS + sid

        @pl.loop(0, _ZR)
        def _(r):
            for j in range(d // 16):
                z_v[r, pl.ds(16 * j, 16)] = jnp.zeros((16,), jnp.float32)

        row0 = (sid * (_ROWS_B // 8) + jnp.minimum(sid, 2)) * 8

        @pl.loop(0, _ROWS_B // _ZR)
        def _(t):
            pltpu.sync_copy(z_v, acc_sh.at[pl.ds(row0 + t * _ZR, _ZR)])

        @pl.when(sid < 2)
        def _():
            pltpu.sync_copy(z_v.at[pl.ds(0, 8)],
                            acc_sh.at[pl.ds(row0 + _ROWS_B, 8)])

        plsc.subcore_barrier()

        def do_chunk(chunk):
            base = chunk * _CH
            pltpu.sync_copy(ei_hbm.at[0, pl.ds(base, _CH)], src_v)
            pltpu.sync_copy(ei_hbm.at[1, pl.ds(base, _CH)], dst_v)
            pltpu.async_copy(table_hbm.at[src_v], rows_v, sem).wait()
            pltpu.sync_copy(rows_v, acc_sh.at[dst_v], add=True)

        @pl.loop(0, _BASE_CHUNKS)
        def _(j):
            do_chunk(w + j * _NW)

        @pl.when(w < _EXTRA)
        def _():
            do_chunk(w + _BASE_CHUNKS * _NW)

        plsc.subcore_barrier()

        @pl.when(sid < 2)
        def _():
            pltpu.sync_copy(acc_sh.at[pl.ds(row0, _ROWS_A)],
                            out_hbm.at[cid, pl.ds(row0, _ROWS_A)])

        @pl.when(sid >= 2)
        def _():
            pltpu.sync_copy(acc_sh.at[pl.ds(row0, _ROWS_B)],
                            out_hbm.at[cid, pl.ds(row0, _ROWS_B)])

    return k(table, edge_index)


def _t1(x, w1p):
    def body(x_ref, w_ref, o_ref):
        col = lax.broadcasted_iota(jnp.int32, (_BM, _D1), 1)
        acc = jnp.dot(x_ref[...], w_ref[...],
                      preferred_element_type=jnp.float32)
        o_ref[...] = acc + (col == 0).astype(jnp.float32)

    return pl.pallas_call(
        body,
        grid=(_N // _BM,),
        in_specs=[pl.BlockSpec((_BM, 128), lambda i: (i, 0)),
                  pl.BlockSpec((128, _D1), lambda i: (0, 0))],
        out_specs=pl.BlockSpec((_BM, _D1), lambda i: (i, 0)),
        out_shape=jax.ShapeDtypeStruct((_N, _D1), jnp.float32),
    )(x, w1p)


def _t2(p1, b1p, w2p):
    def body(p_ref, b_ref, w_ref, o_ref):
        pa = p_ref[0] + p_ref[1]
        deg = jnp.maximum(pa[:, 0:1], 1.0)
        h = jnp.maximum(pa / deg + b_ref[...], 0.0)
        col = lax.broadcasted_iota(jnp.int32, (_BM, _D2), 1)
        o_ref[...] = (jnp.dot(h, w_ref[...],
                              preferred_element_type=jnp.float32)
                      + (col == 0).astype(jnp.float32))

    return pl.pallas_call(
        body,
        grid=(_N // _BM,),
        in_specs=[pl.BlockSpec((_NC, _BM, _D1), lambda i: (0, i, 0)),
                  pl.BlockSpec((1, _D1), lambda i: (0, 0)),
                  pl.BlockSpec((_D1, _D2), lambda i: (0, 0))],
        out_specs=pl.BlockSpec((_BM, _D2), lambda i: (i, 0)),
        out_shape=jax.ShapeDtypeStruct((_N, _D2), jnp.float32),
    )(p1, b1p, w2p)


def _bn(z, g, b):
    m = jnp.mean(z, axis=0, keepdims=True)
    v = jnp.mean((z - m) ** 2, axis=0, keepdims=True)
    return g * (z - m) / jnp.sqrt(v + 1e-5) + b


def _t3(p2, gids, b2p, self_feat, fc1_w, fc1_b, bn1_g, bn1_b,
        fc2_w, fc2_b, bn2_g, bn2_b, fc3_w, fc3_b):
    steps = _N // _BM

    def body(p_ref, g_ref, b2_ref, sf_ref, w1_ref, w1b_ref, g1_ref, bb1_ref,
             w2_ref, w2b_ref, g2_ref, bb2_ref, w3_ref, w3b_ref, o_ref,
             acc_ref):
        i = pl.program_id(0)

        @pl.when(i == 0)
        def _():
            acc_ref[...] = jnp.zeros_like(acc_ref)

        pa = p_ref[0] + p_ref[1]
        deg = jnp.maximum(pa[:, 0:1], 1.0)
        h2 = jnp.maximum(pa / deg + b2_ref[...], 0.0)
        col = lax.broadcasted_iota(jnp.int32, (_BM, _D2), 1)
        h2 = h2 + (col == 0).astype(jnp.float32)
        seg = lax.broadcasted_iota(jnp.int32, (_G, _BM), 0)
        onehot = (g_ref[0] == seg).astype(jnp.float32)
        acc_ref[...] += jnp.dot(onehot, h2,
                                preferred_element_type=jnp.float32)

        @pl.when(i == steps - 1)
        def _():
            acc = acc_ref[...]
            cnt = jnp.maximum(acc[:, 0:1], 1.0)
            hg = acc[:, 1:21] / cnt
            c1 = jnp.concatenate([hg, sf_ref[...]], axis=1)
            z = jnp.dot(c1, w1_ref[...],
                        preferred_element_type=jnp.float32) + w1b_ref[...]
            o1 = jnp.maximum(_bn(z, g1_ref[...], bb1_ref[...]), 0.0)
            c2 = jnp.concatenate([o1, sf_ref[...]], axis=1)
            z2 = jnp.dot(c2, w2_ref[...],
                         preferred_element_type=jnp.float32) + w2b_ref[...]
            o2 = jnp.maximum(_bn(z2, g2_ref[...], bb2_ref[...]), 0.0)
            o_ref[...] = jnp.dot(o2, w3_ref[...],
                                 preferred_element_type=jnp.float32) + w3b_ref[...]

    def full(shape):
        return pl.BlockSpec(shape, lambda i: tuple(0 for _ in shape))

    return pl.pallas_call(
        body,
        grid=(steps,),
        in_specs=[pl.BlockSpec((_NC, _BM, _D2), lambda i: (0, i, 0)),
                  pl.BlockSpec((1, 1, _BM), lambda i: (i, 0, 0)),
                  full((1, _D2)),
                  full((_G, 16)),
                  full((36, 256)), full((1, 256)), full((1, 256)), full((1, 256)),
                  full((272, 32)), full((1, 32)), full((1, 32)), full((1, 32)),
                  full((32, 10)), full((1, 10))],
        out_specs=pl.BlockSpec((_G, 10), lambda i: (0, 0)),
        out_shape=jax.ShapeDtypeStruct((_G, 10), jnp.float32),
        scratch_shapes=[pltpu.VMEM((_G, _D2), jnp.float32)],
    )(p2, gids, b2p, self_feat, fc1_w, fc1_b, bn1_g, bn1_b,
      fc2_w, fc2_b, bn2_g, bn2_b, fc3_w, fc3_b)


def kernel(x, edge_index, graph_ids, self_feat, W1, b1, W2, b2,
           fc1_w, fc1_b, bn1_g, bn1_b, fc2_w, fc2_b, bn2_g, bn2_b,
           fc3_w, fc3_b):
    f32 = jnp.float32
    w1p = jnp.zeros((128, _D1), f32).at[:, 1:101].set(W1)
    b1p = jnp.zeros((1, _D1), f32).at[0, 0].set(-1.0).at[0, 1:101].set(b1)
    w2p = jnp.zeros((_D1, _D2), f32).at[1:101, 1:21].set(W2)
    b2p = jnp.zeros((1, _D2), f32).at[0, 0].set(-1.0).at[0, 1:21].set(b2)

    t1 = _t1(x, w1p)
    p1 = _sc_mean_agg(t1, edge_index, _D1)
    t2 = _t2(p1, b1p, w2p)
    p2 = _sc_mean_agg(t2, edge_index, _D2)
    gids = graph_ids.reshape(_N // _BM, 1, _BM)
    return _t3(p2, gids, b2p, self_feat,
               fc1_w, fc1_b.reshape(1, -1), bn1_g.reshape(1, -1),
               bn1_b.reshape(1, -1), fc2_w, fc2_b.reshape(1, -1),
               bn2_g.reshape(1, -1), bn2_b.reshape(1, -1),
               fc3_w, fc3_b.reshape(1, -1))


# speedup vs baseline: 14.8717x; 1.7896x over previous
import functools

import jax
import jax.numpy as jnp
from jax import lax
from jax.experimental import pallas as pl
from jax.experimental.pallas import tpu as pltpu
from jax.experimental.pallas import tpu_sc as plsc

_N = 10000
_E = 320000
_G = 64
_D1 = 112
_D2 = 32

_NC, _NS = 2, 16
_NW = _NC * _NS
_CH = 128
_NCHUNKS = _E // _CH
_BASE_CHUNKS = _NCHUNKS // _NW
_EXTRA = _NCHUNKS % _NW
_ROWS_A = 632
_ROWS_B = 624
_ZR = 104

_BM = 1000


def _sc_mean_agg(table, edge_index, d):
    mesh = plsc.VectorSubcoreMesh(core_axis_name="c", subcore_axis_name="s")

    @functools.partial(
        pl.kernel,
        mesh=mesh,
        compiler_params=pltpu.CompilerParams(use_tc_tiling_on_sc=False),
        out_type=jax.ShapeDtypeStruct((_NC, _N, d), jnp.float32),
        scratch_types=[
            pltpu.VMEM((2, _CH), jnp.int32),
            pltpu.VMEM((2, _CH), jnp.int32),
            pltpu.VMEM((_CH, d), jnp.float32),
            pltpu.VMEM((_CH, d), jnp.float32),
            pltpu.VMEM((_ZR, d), jnp.float32),
            pltpu.VMEM_SHARED((_N, d), jnp.float32),
            pltpu.SemaphoreType.DMA,
            pltpu.SemaphoreType.DMA,
        ],
    )
    def k(table_hbm, ei_hbm, out_hbm, i0_v, i1_v, r0_v, r1_v, z_v, acc_sh,
          sem0, sem1):
        cid = lax.axis_index("c")
        sid = lax.axis_index("s")
        w = cid * _NS + sid

        @pl.loop(0, _ZR)
        def _(r):
            for j in range(d // 16):
                z_v[r, pl.ds(16 * j, 16)] = jnp.zeros((16,), jnp.float32)

        row0 = (sid * (_ROWS_B // 8) + jnp.minimum(sid, 2)) * 8

        @pl.loop(0, _ROWS_B // _ZR)
        def _(t):
            pltpu.sync_copy(z_v, acc_sh.at[pl.ds(row0 + t * _ZR, _ZR)])

        @pl.when(sid < 2)
        def _():
            pltpu.sync_copy(z_v.at[pl.ds(0, 8)],
                            acc_sh.at[pl.ds(row0 + _ROWS_B, 8)])

        plsc.subcore_barrier()

        def load_idx(i_v, j):
            pltpu.sync_copy(ei_hbm.at[:, pl.ds((w + j * _NW) * _CH, _CH)], i_v)

        def start_gather(i_v, r_v, sem):
            pltpu.async_copy(table_hbm.at[i_v.at[0]], r_v, sem)

        def wait_gather(i_v, r_v, sem):
            pltpu.make_async_copy(table_hbm.at[i_v.at[0]], r_v, sem).wait()

        def scatter(i_v, r_v):
            pltpu.sync_copy(r_v, acc_sh.at[i_v.at[1]], add=True)

        load_idx(i0_v, 0)
        start_gather(i0_v, r0_v, sem0)
        load_idx(i1_v, 1)
        start_gather(i1_v, r1_v, sem1)

        half = _BASE_CHUNKS // 2

        @pl.loop(0, half)
        def _(t):
            wait_gather(i0_v, r0_v, sem0)
            scatter(i0_v, r0_v)

            @pl.when(t < half - 1)
            def _():
                load_idx(i0_v, 2 * t + 2)
                start_gather(i0_v, r0_v, sem0)

            wait_gather(i1_v, r1_v, sem1)
            scatter(i1_v, r1_v)

            @pl.when(t < half - 1)
            def _():
                load_idx(i1_v, 2 * t + 3)
                start_gather(i1_v, r1_v, sem1)

        @pl.when(w < _EXTRA)
        def _():
            load_idx(i0_v, _BASE_CHUNKS)
            start_gather(i0_v, r0_v, sem0)
            wait_gather(i0_v, r0_v, sem0)
            scatter(i0_v, r0_v)

        plsc.subcore_barrier()

        @pl.when(sid < 2)
        def _():
            pltpu.sync_copy(acc_sh.at[pl.ds(row0, _ROWS_A)],
                            out_hbm.at[cid, pl.ds(row0, _ROWS_A)])

        @pl.when(sid >= 2)
        def _():
            pltpu.sync_copy(acc_sh.at[pl.ds(row0, _ROWS_B)],
                            out_hbm.at[cid, pl.ds(row0, _ROWS_B)])

    return k(table, edge_index)


def _t1(x, w1p):
    def body(x_ref, w_ref, o_ref):
        col = lax.broadcasted_iota(jnp.int32, (_BM, _D1), 1)
        acc = jnp.dot(x_ref[...], w_ref[...],
                      preferred_element_type=jnp.float32)
        o_ref[...] = acc + (col == 0).astype(jnp.float32)

    return pl.pallas_call(
        body,
        grid=(_N // _BM,),
        in_specs=[pl.BlockSpec((_BM, 128), lambda i: (i, 0)),
                  pl.BlockSpec((128, _D1), lambda i: (0, 0))],
        out_specs=pl.BlockSpec((_BM, _D1), lambda i: (i, 0)),
        out_shape=jax.ShapeDtypeStruct((_N, _D1), jnp.float32),
    )(x, w1p)


def _t2(p1, b1p, w2p):
    def body(p_ref, b_ref, w_ref, o_ref):
        pa = p_ref[0] + p_ref[1]
        deg = jnp.maximum(pa[:, 0:1], 1.0)
        h = jnp.maximum(pa / deg + b_ref[...], 0.0)
        col = lax.broadcasted_iota(jnp.int32, (_BM, _D2), 1)
        o_ref[...] = (jnp.dot(h, w_ref[...],
                              preferred_element_type=jnp.float32)
                      + (col == 0).astype(jnp.float32))

    return pl.pallas_call(
        body,
        grid=(_N // _BM,),
        in_specs=[pl.BlockSpec((_NC, _BM, _D1), lambda i: (0, i, 0)),
                  pl.BlockSpec((1, _D1), lambda i: (0, 0)),
                  pl.BlockSpec((_D1, _D2), lambda i: (0, 0))],
        out_specs=pl.BlockSpec((_BM, _D2), lambda i: (i, 0)),
        out_shape=jax.ShapeDtypeStruct((_N, _D2), jnp.float32),
    )(p1, b1p, w2p)


def _bn(z, g, b):
    m = jnp.mean(z, axis=0, keepdims=True)
    v = jnp.mean((z - m) ** 2, axis=0, keepdims=True)
    return g * (z - m) / jnp.sqrt(v + 1e-5) + b


def _t3(p2, gids, b2p, self_feat, fc1_w, fc1_b, bn1_g, bn1_b,
        fc2_w, fc2_b, bn2_g, bn2_b, fc3_w, fc3_b):
    steps = _N // _BM

    def body(p_ref, g_ref, b2_ref, sf_ref, w1_ref, w1b_ref, g1_ref, bb1_ref,
             w2_ref, w2b_ref, g2_ref, bb2_ref, w3_ref, w3b_ref, o_ref,
             acc_ref):
        i = pl.program_id(0)

        @pl.when(i == 0)
        def _():
            acc_ref[...] = jnp.zeros_like(acc_ref)

        pa = p_ref[0] + p_ref[1]
        deg = jnp.maximum(pa[:, 0:1], 1.0)
        h2 = jnp.maximum(pa / deg + b2_ref[...], 0.0)
        col = lax.broadcasted_iota(jnp.int32, (_BM, _D2), 1)
        h2 = h2 + (col == 0).astype(jnp.float32)
        seg = lax.broadcasted_iota(jnp.int32, (_G, _BM), 0)
        onehot = (g_ref[0] == seg).astype(jnp.float32)
        acc_ref[...] += jnp.dot(onehot, h2,
                                preferred_element_type=jnp.float32)

        @pl.when(i == steps - 1)
        def _():
            acc = acc_ref[...]
            cnt = jnp.maximum(acc[:, 0:1], 1.0)
            hg = acc[:, 1:21] / cnt
            c1 = jnp.concatenate([hg, sf_ref[...]], axis=1)
            z = jnp.dot(c1, w1_ref[...],
                        preferred_element_type=jnp.float32) + w1b_ref[...]
            o1 = jnp.maximum(_bn(z, g1_ref[...], bb1_ref[...]), 0.0)
            c2 = jnp.concatenate([o1, sf_ref[...]], axis=1)
            z2 = jnp.dot(c2, w2_ref[...],
                         preferred_element_type=jnp.float32) + w2b_ref[...]
            o2 = jnp.maximum(_bn(z2, g2_ref[...], bb2_ref[...]), 0.0)
            o_ref[...] = jnp.dot(o2, w3_ref[...],
                                 preferred_element_type=jnp.float32) + w3b_ref[...]

    def full(shape):
        return pl.BlockSpec(shape, lambda i: tuple(0 for _ in shape))

    return pl.pallas_call(
        body,
        grid=(steps,),
        in_specs=[pl.BlockSpec((_NC, _BM, _D2), lambda i: (0, i, 0)),
                  pl.BlockSpec((1, 1, _BM), lambda i: (i, 0, 0)),
                  full((1, _D2)),
                  full((_G, 16)),
                  full((36, 256)), full((1, 256)), full((1, 256)), full((1, 256)),
                  full((272, 32)), full((1, 32)), full((1, 32)), full((1, 32)),
                  full((32, 10)), full((1, 10))],
        out_specs=pl.BlockSpec((_G, 10), lambda i: (0, 0)),
        out_shape=jax.ShapeDtypeStruct((_G, 10), jnp.float32),
        scratch_shapes=[pltpu.VMEM((_G, _D2), jnp.float32)],
    )(p2, gids, b2p, self_feat, fc1_w, fc1_b, bn1_g, bn1_b,
      fc2_w, fc2_b, bn2_g, bn2_b, fc3_w, fc3_b)


def kernel(x, edge_index, graph_ids, self_feat, W1, b1, W2, b2,
           fc1_w, fc1_b, bn1_g, bn1_b, fc2_w, fc2_b, bn2_g, bn2_b,
           fc3_w, fc3_b):
    f32 = jnp.float32
    w1p = jnp.zeros((128, _D1), f32).at[:, 1:101].set(W1)
    b1p = jnp.zeros((1, _D1), f32).at[0, 0].set(-1.0).at[0, 1:101].set(b1)
    w2p = jnp.zeros((_D1, _D2), f32).at[1:101, 1:21].set(W2)
    b2p = jnp.zeros((1, _D2), f32).at[0, 0].set(-1.0).at[0, 1:21].set(b2)

    t1 = _t1(x, w1p)
    p1 = _sc_mean_agg(t1, edge_index, _D1)
    t2 = _t2(p1, b1p, w2p)
    p2 = _sc_mean_agg(t2, edge_index, _D2)
    gids = graph_ids.reshape(_N // _BM, 1, _BM)
    return _t3(p2, gids, b2p, self_feat,
               fc1_w, fc1_b.reshape(1, -1), bn1_g.reshape(1, -1),
               bn1_b.reshape(1, -1), fc2_w, fc2_b.reshape(1, -1),
               bn2_g.reshape(1, -1), bn2_b.reshape(1, -1),
               fc3_w, fc3_b.reshape(1, -1))


# speedup vs baseline: 16.2730x; 1.0942x over previous
import functools

import jax
import jax.numpy as jnp
from jax import lax
from jax.experimental import pallas as pl
from jax.experimental.pallas import tpu as pltpu
from jax.experimental.pallas import tpu_sc as plsc

_N = 10000
_E = 320000
_G = 64
_D1 = 112
_D2 = 32

_NC, _NS = 2, 16
_NW = _NC * _NS
_CH = 128
_NCHUNKS = _E // _CH
_BASE_CHUNKS = _NCHUNKS // _NW
_EXTRA = _NCHUNKS % _NW
_SB = 13
_NSB = _BASE_CHUNKS // _SB
_ROWS_A = 632
_ROWS_B = 624
_ZR = 104

_BM = 1000


def _sc_mean_agg(table, ei3, d):
    mesh = plsc.VectorSubcoreMesh(core_axis_name="c", subcore_axis_name="s")

    @functools.partial(
        pl.kernel,
        mesh=mesh,
        compiler_params=pltpu.CompilerParams(use_tc_tiling_on_sc=False),
        out_type=jax.ShapeDtypeStruct((_NC, _N, d), jnp.float32),
        scratch_types=[
            pltpu.VMEM((_SB, _CH), jnp.int32),
            pltpu.VMEM((_SB, _CH), jnp.int32),
            pltpu.VMEM((_SB, _CH), jnp.int32),
            pltpu.VMEM((_SB, _CH), jnp.int32),
            pltpu.VMEM((_CH, d), jnp.float32),
            pltpu.VMEM((_CH, d), jnp.float32),
            pltpu.VMEM((_ZR, d), jnp.float32),
            pltpu.VMEM_SHARED((_N, d), jnp.float32),
            pltpu.SemaphoreType.DMA,
            pltpu.SemaphoreType.DMA,
            pltpu.SemaphoreType.DMA,
        ],
    )
    def k(table_hbm, ei_hbm, out_hbm, sA_v, dA_v, sB_v, dB_v, r0_v, r1_v,
          z_v, acc_sh, sem0, sem1, isem):
        cid = lax.axis_index("c")
        sid = lax.axis_index("s")
        w = cid * _NS + sid

        @pl.loop(0, _ZR)
        def _(r):
            for j in range(d // 16):
                z_v[r, pl.ds(16 * j, 16)] = jnp.zeros((16,), jnp.float32)

        row0 = (sid * (_ROWS_B // 8) + jnp.minimum(sid, 2)) * 8

        @pl.loop(0, _ROWS_B // _ZR)
        def _(t):
            pltpu.sync_copy(z_v, acc_sh.at[pl.ds(row0 + t * _ZR, _ZR)])

        @pl.when(sid < 2)
        def _():
            pltpu.sync_copy(z_v.at[pl.ds(0, 8)],
                            acc_sh.at[pl.ds(row0 + _ROWS_B, 8)])

        plsc.subcore_barrier()

        cbase = _BASE_CHUNKS * w + jnp.minimum(w, _EXTRA)
        rows = (r0_v, r1_v)
        gsems = (sem0, sem1)

        def prefetch_idx(s_v, d_v, sb):
            c0 = cbase + sb * _SB
            pltpu.async_copy(ei_hbm.at[0, pl.ds(c0, _SB)], s_v, isem)
            pltpu.async_copy(ei_hbm.at[1, pl.ds(c0, _SB)], d_v, isem)

        def wait_idx(s_v, d_v, sb):
            c0 = cbase + sb * _SB
            pltpu.make_async_copy(ei_hbm.at[0, pl.ds(c0, _SB)], s_v,
                                  isem).wait()
            pltpu.make_async_copy(ei_hbm.at[1, pl.ds(c0, _SB)], d_v,
                                  isem).wait()

        def start_gather(s_v, j, b):
            pltpu.async_copy(table_hbm.at[s_v.at[j]], rows[b], gsems[b])

        def wait_gather(s_v, j, b):
            pltpu.make_async_copy(table_hbm.at[s_v.at[j]], rows[b],
                                  gsems[b]).wait()

        def scatter(d_v, j, b):
            pltpu.sync_copy(rows[b], acc_sh.at[d_v.at[j]], add=True)

        def run_superblock(s_v, d_v, s_next, d_next, sb):
            wait_idx(s_v, d_v, sb)
            start_gather(s_v, 0, 0)
            start_gather(s_v, 1, 1)

            @pl.when(sb + 1 < _NSB)
            def _():
                prefetch_idx(s_next, d_next, sb + 1)

            for j in range(_SB):
                b = j % 2
                wait_gather(s_v, j, b)
                scatter(d_v, j, b)
                if j + 2 < _SB:
                    start_gather(s_v, j + 2, b)

        prefetch_idx(sA_v, dA_v, 0)

        @pl.loop(0, _NSB // 2)
        def _(u):
            run_superblock(sA_v, dA_v, sB_v, dB_v, 2 * u)
            run_superblock(sB_v, dB_v, sA_v, dA_v, 2 * u + 1)

        @pl.when(w < _EXTRA)
        def _():
            c0 = cbase + _BASE_CHUNKS
            pltpu.sync_copy(ei_hbm.at[0, pl.ds(c0, 1)], sA_v.at[pl.ds(0, 1)])
            pltpu.sync_copy(ei_hbm.at[1, pl.ds(c0, 1)], dA_v.at[pl.ds(0, 1)])
            start_gather(sA_v, 0, 0)
            wait_gather(sA_v, 0, 0)
            scatter(dA_v, 0, 0)

        plsc.subcore_barrier()

        @pl.when(sid < 2)
        def _():
            pltpu.sync_copy(acc_sh.at[pl.ds(row0, _ROWS_A)],
                            out_hbm.at[cid, pl.ds(row0, _ROWS_A)])

        @pl.when(sid >= 2)
        def _():
            pltpu.sync_copy(acc_sh.at[pl.ds(row0, _ROWS_B)],
                            out_hbm.at[cid, pl.ds(row0, _ROWS_B)])

    return k(table, ei3)


def _t1(x, w1p):
    def body(x_ref, w_ref, o_ref):
        col = lax.broadcasted_iota(jnp.int32, (_BM, _D1), 1)
        acc = jnp.dot(x_ref[...], w_ref[...],
                      preferred_element_type=jnp.float32)
        o_ref[...] = acc + (col == 0).astype(jnp.float32)

    return pl.pallas_call(
        body,
        grid=(_N // _BM,),
        in_specs=[pl.BlockSpec((_BM, 128), lambda i: (i, 0)),
                  pl.BlockSpec((128, _D1), lambda i: (0, 0))],
        out_specs=pl.BlockSpec((_BM, _D1), lambda i: (i, 0)),
        out_shape=jax.ShapeDtypeStruct((_N, _D1), jnp.float32),
    )(x, w1p)


def _t2(p1, b1p, w2p):
    def body(p_ref, b_ref, w_ref, o_ref):
        pa = p_ref[0] + p_ref[1]
        deg = jnp.maximum(pa[:, 0:1], 1.0)
        h = jnp.maximum(pa / deg + b_ref[...], 0.0)
        col = lax.broadcasted_iota(jnp.int32, (_BM, _D2), 1)
        o_ref[...] = (jnp.dot(h, w_ref[...],
                              preferred_element_type=jnp.float32)
                      + (col == 0).astype(jnp.float32))

    return pl.pallas_call(
        body,
        grid=(_N // _BM,),
        in_specs=[pl.BlockSpec((_NC, _BM, _D1), lambda i: (0, i, 0)),
                  pl.BlockSpec((1, _D1), lambda i: (0, 0)),
                  pl.BlockSpec((_D1, _D2), lambda i: (0, 0))],
        out_specs=pl.BlockSpec((_BM, _D2), lambda i: (i, 0)),
        out_shape=jax.ShapeDtypeStruct((_N, _D2), jnp.float32),
    )(p1, b1p, w2p)


def _bn(z, g, b):
    m = jnp.mean(z, axis=0, keepdims=True)
    v = jnp.mean((z - m) ** 2, axis=0, keepdims=True)
    return g * (z - m) / jnp.sqrt(v + 1e-5) + b


def _t3(p2, gids, b2p, self_feat, fc1_w, fc1_b, bn1_g, bn1_b,
        fc2_w, fc2_b, bn2_g, bn2_b, fc3_w, fc3_b):
    steps = _N // _BM

    def body(p_ref, g_ref, b2_ref, sf_ref, w1_ref, w1b_ref, g1_ref, bb1_ref,
             w2_ref, w2b_ref, g2_ref, bb2_ref, w3_ref, w3b_ref, o_ref,
             acc_ref):
        i = pl.program_id(0)

        @pl.when(i == 0)
        def _():
            acc_ref[...] = jnp.zeros_like(acc_ref)

        pa = p_ref[0] + p_ref[1]
        deg = jnp.maximum(pa[:, 0:1], 1.0)
        h2 = jnp.maximum(pa / deg + b2_ref[...], 0.0)
        col = lax.broadcasted_iota(jnp.int32, (_BM, _D2), 1)
        h2 = h2 + (col == 0).astype(jnp.float32)
        seg = lax.broadcasted_iota(jnp.int32, (_G, _BM), 0)
        onehot = (g_ref[0] == seg).astype(jnp.float32)
        acc_ref[...] += jnp.dot(onehot, h2,
                                preferred_element_type=jnp.float32)

        @pl.when(i == steps - 1)
        def _():
            acc = acc_ref[...]
            cnt = jnp.maximum(acc[:, 0:1], 1.0)
            hg = acc[:, 1:21] / cnt
            c1 = jnp.concatenate([hg, sf_ref[...]], axis=1)
            z = jnp.dot(c1, w1_ref[...],
                        preferred_element_type=jnp.float32) + w1b_ref[...]
            o1 = jnp.maximum(_bn(z, g1_ref[...], bb1_ref[...]), 0.0)
            c2 = jnp.concatenate([o1, sf_ref[...]], axis=1)
            z2 = jnp.dot(c2, w2_ref[...],
                         preferred_element_type=jnp.float32) + w2b_ref[...]
            o2 = jnp.maximum(_bn(z2, g2_ref[...], bb2_ref[...]), 0.0)
            o_ref[...] = jnp.dot(o2, w3_ref[...],
                                 preferred_element_type=jnp.float32) + w3b_ref[...]

    def full(shape):
        return pl.BlockSpec(shape, lambda i: tuple(0 for _ in shape))

    return pl.pallas_call(
        body,
        grid=(steps,),
        in_specs=[pl.BlockSpec((_NC, _BM, _D2), lambda i: (0, i, 0)),
                  pl.BlockSpec((1, 1, _BM), lambda i: (i, 0, 0)),
                  full((1, _D2)),
                  full((_G, 16)),
                  full((36, 256)), full((1, 256)), full((1, 256)), full((1, 256)),
                  full((272, 32)), full((1, 32)), full((1, 32)), full((1, 32)),
                  full((32, 10)), full((1, 10))],
        out_specs=pl.BlockSpec((_G, 10), lambda i: (0, 0)),
        out_shape=jax.ShapeDtypeStruct((_G, 10), jnp.float32),
        scratch_shapes=[pltpu.VMEM((_G, _D2), jnp.float32)],
    )(p2, gids, b2p, self_feat, fc1_w, fc1_b, bn1_g, bn1_b,
      fc2_w, fc2_b, bn2_g, bn2_b, fc3_w, fc3_b)


def kernel(x, edge_index, graph_ids, self_feat, W1, b1, W2, b2,
           fc1_w, fc1_b, bn1_g, bn1_b, fc2_w, fc2_b, bn2_g, bn2_b,
           fc3_w, fc3_b):
    f32 = jnp.float32
    w1p = jnp.zeros((128, _D1), f32).at[:, 1:101].set(W1)
    b1p = jnp.zeros((1, _D1), f32).at[0, 0].set(-1.0).at[0, 1:101].set(b1)
    w2p = jnp.zeros((_D1, _D2), f32).at[1:101, 1:21].set(W2)
    b2p = jnp.zeros((1, _D2), f32).at[0, 0].set(-1.0).at[0, 1:21].set(b2)

    ei3 = edge_index.reshape(2, _NCHUNKS, _CH)
    t1 = _t1(x, w1p)
    p1 = _sc_mean_agg(t1, ei3, _D1)
    t2 = _t2(p1, b1p, w2p)
    p2 = _sc_mean_agg(t2, ei3, _D2)
    gids = graph_ids.reshape(_N // _BM, 1, _BM)
    return _t3(p2, gids, b2p, self_feat,
               fc1_w, fc1_b.reshape(1, -1), bn1_g.reshape(1, -1),
               bn1_b.reshape(1, -1), fc2_w, fc2_b.reshape(1, -1),
               bn2_g.reshape(1, -1), bn2_b.reshape(1, -1),
               fc3_w, fc3_b.reshape(1, -1))
